# baseline (speedup 1.0000x reference)
import functools

import jax
import jax.numpy as jnp
from jax.experimental import pallas as pl
from jax.experimental.pallas import tpu as pltpu


def _lcam_kernel(x_ref, w1_ref, w2_ref, o_ref, *, n_batch, c):
    w1 = w1_ref[...]
    w2 = w2_ref[...]
    inv_hw = 1.0 / x_ref.shape[-1]
    for b in range(n_batch):
        xb = x_ref[pl.ds(b * c, c), :]
        pmax = jnp.max(xb, axis=-1, keepdims=True)
        pavg = jnp.sum(xb, axis=-1, keepdims=True) * inv_hw
        p2 = jnp.concatenate([pmax, pavg], axis=1)
        h = jnp.maximum(
            jnp.dot(w1, p2, preferred_element_type=jnp.float32), 0.0)
        y = jnp.dot(w2, h[:, 0:1] + h[:, 1:2],
                    preferred_element_type=jnp.float32)
        o_ref[pl.ds(b * c, c), :] = jax.nn.sigmoid(y)


@jax.jit
def _lcam(x, w1, w2):
    B, C, H, W = x.shape
    C_ = w1.shape[0]
    HW = H * W
    R = B * C

    x2 = x.reshape(R, HW)
    w1m = w1.reshape(C_, C).astype(jnp.float32)
    w2m = w2.reshape(C, C_).astype(jnp.float32)

    n_batch = 1
    grid = (B // n_batch,)
    row_blk = n_batch * C

    out = pl.pallas_call(
        functools.partial(_lcam_kernel, n_batch=n_batch, c=C),
        out_shape=jax.ShapeDtypeStruct((R, 1), jnp.float32),
        grid=grid,
        in_specs=[
            pl.BlockSpec((row_blk, HW), lambda i: (i, 0)),
            pl.BlockSpec((C_, C), lambda i: (0, 0)),
            pl.BlockSpec((C, C_), lambda i: (0, 0)),
        ],
        out_specs=pl.BlockSpec((row_blk, 1), lambda i: (i, 0)),
        compiler_params=pltpu.CompilerParams(
            dimension_semantics=("parallel",),
            vmem_limit_bytes=64 * 1024 * 1024),
    )(x2.astype(jnp.float32), w1m, w2m)

    return out.reshape(B, C, 1, 1).astype(x.dtype)


def kernel(x, w1, w2):
    return _lcam(x, w1, w2)
